# baseline (speedup 1.0000x reference)
import functools

import jax
import jax.numpy as jnp
from jax import lax
from jax.experimental import pallas as pl
from jax.experimental.pallas import tpu as pltpu
from jax.experimental.pallas import tpu_sc as plsc

N_NODES = 10000
D = 128
N_EDGES = 320000

NC = 2
NS = 16
NW = NC * NS
LANES = 16

CHUNK = 48
NCHUNK = 216
PER_W = NCHUNK * CHUNK
E_PAD = NW * PER_W
ROWS_PER_TILE = N_NODES // NS
GRP = CHUNK // LANES

_SC_PARAMS = pltpu.CompilerParams(needs_layout_passes=False,
                                  use_tc_tiling_on_sc=False)


def _mesh():
    return plsc.VectorSubcoreMesh(core_axis_name="c", subcore_axis_name="s",
                                  num_cores=NC, num_subcores=NS)


def _wid():
    return lax.axis_index("s") * NC + lax.axis_index("c")


@functools.partial(
    pl.kernel,
    out_type=jax.ShapeDtypeStruct((NW, N_NODES), jnp.float32),
    mesh=_mesh(),
    compiler_params=_SC_PARAMS,
    scratch_types=[
        pltpu.VMEM((NCHUNK, CHUNK), jnp.int32),
        pltpu.VMEM((NCHUNK, CHUNK), jnp.float32),
        pltpu.VMEM((N_NODES,), jnp.float32),
    ],
)
def _deg_kernel(col_hbm, ew_hbm, out_hbm, col_v, ew_v, deg_v):
    w = _wid()
    pltpu.sync_copy(col_hbm.at[w], col_v)
    pltpu.sync_copy(ew_hbm.at[w], ew_v)

    def zbody(i, _):
        deg_v[pl.ds(i * LANES, LANES)] = jnp.zeros((LANES,), jnp.float32)
        return 0

    lax.fori_loop(0, N_NODES // LANES, zbody, 0)

    def ebody(t, _):
        j = t // GRP
        k = t % GRP
        sl = pl.ds(k * LANES, LANES)
        plsc.addupdate_scatter(deg_v, [col_v[j, sl]], ew_v[j, sl])
        return 0

    lax.fori_loop(0, NCHUNK * GRP, ebody, 0)
    pltpu.sync_copy(deg_v, out_hbm.at[w])


def _tc_pre_body(x_ref, w1_ref, degp_ref, h_ref, dis_ref):
    deg = jnp.sum(degp_ref[...], axis=0, keepdims=True)
    dis_ref[...] = lax.rsqrt(deg)
    h_ref[...] = jnp.dot(x_ref[...], w1_ref[...],
                         preferred_element_type=jnp.float32)


_tc_pre = pl.pallas_call(
    _tc_pre_body,
    out_shape=(
        jax.ShapeDtypeStruct((N_NODES, D), jnp.float32),
        jax.ShapeDtypeStruct((1, N_NODES), jnp.float32),
    ),
)


@functools.partial(
    pl.kernel,
    out_type=jax.ShapeDtypeStruct((NW, NCHUNK, CHUNK), jnp.float32),
    mesh=_mesh(),
    compiler_params=_SC_PARAMS,
    scratch_types=[
        pltpu.VMEM((NCHUNK, CHUNK), jnp.int32),
        pltpu.VMEM((NCHUNK, CHUNK), jnp.int32),
        pltpu.VMEM((NCHUNK, CHUNK), jnp.float32),
        pltpu.VMEM((N_NODES,), jnp.float32),
    ],
)
def _norm_kernel(row_hbm, col_hbm, ew_hbm, dis_hbm, out_hbm,
                 row_v, col_v, nrm_v, dis_v):
    w = _wid()
    pltpu.sync_copy(row_hbm.at[w], row_v)
    pltpu.sync_copy(col_hbm.at[w], col_v)
    pltpu.sync_copy(ew_hbm.at[w], nrm_v)
    pltpu.sync_copy(dis_hbm, dis_v)

    def nbody(t, _):
        j = t // GRP
        k = t % GRP
        sl = pl.ds(k * LANES, LANES)
        dr = plsc.load_gather(dis_v, [row_v[j, sl]])
        dc = plsc.load_gather(dis_v, [col_v[j, sl]])
        nrm_v[j, sl] = dr * nrm_v[j, sl] * dc
        return 0

    lax.fori_loop(0, NCHUNK * GRP, nbody, 0)
    pltpu.sync_copy(nrm_v, out_hbm.at[w])


def _layer_body(h_hbm, row_hbm, col_hbm, nrm_hbm, part_hbm,
                row_v, col_v, nrm_v, gb0, gb1, gb2, acc,
                sh0, sh1, sh2, ss0, ss1, ss2):
    core = lax.axis_index("c")
    sub = lax.axis_index("s")
    w = sub * NC + core

    pltpu.sync_copy(row_hbm.at[w], row_v)
    pltpu.sync_copy(col_hbm.at[w], col_v)
    pltpu.sync_copy(nrm_hbm.at[w], nrm_v)

    def zb(i, _):
        gb0[i // (D // LANES), pl.ds((i % (D // LANES)) * LANES, LANES)] = (
            jnp.zeros((LANES,), jnp.float32))
        return 0

    lax.fori_loop(0, CHUNK * (D // LANES), zb, 0)
    zbase = sub * ROWS_PER_TILE
    for z in range(ROWS_PER_TILE // CHUNK):
        pltpu.sync_copy(gb0, acc.at[pl.ds(zbase + z * CHUNK, CHUNK)])
    zrem = ROWS_PER_TILE % CHUNK
    if zrem:
        pltpu.sync_copy(
            gb0.at[pl.ds(0, zrem)],
            acc.at[pl.ds(zbase + (ROWS_PER_TILE // CHUNK) * CHUNK, zrem)])
    plsc.subcore_barrier()

    def scale(gb, g):
        @plsc.parallel_loop(0, CHUNK, 1, unroll=8)
        def _(e):
            nv = plsc.load_gather(nrm_v.at[g],
                                  [jnp.full((LANES,), e, jnp.int32)])
            for b in range(D // LANES):
                sl = pl.ds(b * LANES, LANES)
                gb[e, sl] = gb[e, sl] * nv

    gbs = (gb0, gb1, gb2)
    shs = (sh0, sh1, sh2)
    sss = (ss0, ss1, ss2)

    pltpu.async_copy(h_hbm.at[row_v.at[0]], gb0, sh0)
    pltpu.async_copy(h_hbm.at[row_v.at[1]], gb1, sh1)

    def tri(i, _):
        for k in range(3):
            c = i * 3 + k
            gb, sh, ss = gbs[k], shs[k], sss[k]
            kp = (k + 2) % 3
            pltpu.make_async_copy(h_hbm.at[row_v.at[c]], gb, sh).wait()
            scale(gb, c)
            pltpu.async_copy(gb, acc.at[col_v.at[c]], sss[k], add=True)

            def drain_prev():
                pltpu.make_async_copy(
                    gbs[kp], acc.at[col_v.at[c - 1]], sss[kp]).wait()

            if k == 0:
                pl.when(i > 0)(drain_prev)
            else:
                drain_prev()

            @pl.when(c + 2 < NCHUNK)
            def _():
                pltpu.async_copy(h_hbm.at[row_v.at[c + 2]], gbs[kp], shs[kp])

        return 0

    lax.fori_loop(0, NCHUNK // 3, tri, 0)
    pltpu.make_async_copy(gbs[(NCHUNK - 1) % 3],
                          acc.at[col_v.at[NCHUNK - 1]],
                          sss[(NCHUNK - 1) % 3]).wait()
    plsc.subcore_barrier()

    rbase = sub * ROWS_PER_TILE
    pltpu.sync_copy(acc.at[pl.ds(rbase, ROWS_PER_TILE)],
                    part_hbm.at[core, pl.ds(rbase, ROWS_PER_TILE)])


_layer = pl.kernel(
    _layer_body,
    out_type=jax.ShapeDtypeStruct((NC, N_NODES, D), jnp.float32),
    mesh=_mesh(),
    compiler_params=_SC_PARAMS,
    scratch_types=[
        pltpu.VMEM((NCHUNK, CHUNK), jnp.int32),
        pltpu.VMEM((NCHUNK, CHUNK), jnp.int32),
        pltpu.VMEM((NCHUNK, CHUNK), jnp.float32),
        pltpu.VMEM((CHUNK, D), jnp.float32),
        pltpu.VMEM((CHUNK, D), jnp.float32),
        pltpu.VMEM((CHUNK, D), jnp.float32),
        pltpu.VMEM_SHARED((N_NODES, D), jnp.float32),
        pltpu.SemaphoreType.DMA,
        pltpu.SemaphoreType.DMA,
        pltpu.SemaphoreType.DMA,
        pltpu.SemaphoreType.DMA,
        pltpu.SemaphoreType.DMA,
        pltpu.SemaphoreType.DMA,
    ],
)


def _tc_mid_body(p_ref, b_ref, w2_ref, out_ref):
    s = p_ref[0] + p_ref[1] + b_ref[...]
    out_ref[...] = jnp.dot(jnp.maximum(s, 0.0), w2_ref[...],
                           preferred_element_type=jnp.float32)


_tc_mid = pl.pallas_call(
    _tc_mid_body,
    out_shape=jax.ShapeDtypeStruct((N_NODES, D), jnp.float32),
)


def _tc_post_body(p_ref, b_ref, out_ref):
    out_ref[...] = p_ref[0] + p_ref[1] + b_ref[...]


_tc_post = pl.pallas_call(
    _tc_post_body,
    out_shape=jax.ShapeDtypeStruct((N_NODES, D), jnp.float32),
)


def kernel(x, edge_index, edge_weight, W1, b1, W2, b2):
    row = edge_index[0].astype(jnp.int32)
    col = edge_index[1].astype(jnp.int32)
    ew = edge_weight.astype(jnp.float32)

    loop = jnp.arange(N_NODES, dtype=jnp.int32)
    pad = E_PAD - (N_EDGES + N_NODES)
    pad_i = jnp.arange(pad, dtype=jnp.int32)
    zpad_f = jnp.zeros((pad,), jnp.float32)
    row3 = jnp.concatenate([row, loop, pad_i]).reshape(NW, NCHUNK, CHUNK)
    col3 = jnp.concatenate([col, loop, pad_i]).reshape(NW, NCHUNK, CHUNK)
    ew3 = jnp.concatenate(
        [ew, jnp.ones((N_NODES,), jnp.float32), zpad_f]
    ).reshape(NW, NCHUNK, CHUNK)

    degp = _deg_kernel(col3, ew3)
    h1, dis2d = _tc_pre(x, W1, degp)
    dis = dis2d.reshape(N_NODES)
    norm3 = _norm_kernel(row3, col3, ew3, dis)

    p1 = _layer(h1, row3, col3, norm3)
    h2 = _tc_mid(p1, b1.reshape(1, D), W2)

    p2 = _layer(h2, row3, col3, norm3)
    out = _tc_post(p2, b2.reshape(1, D))
    return out


# speedup vs baseline: 1.1441x; 1.1441x over previous
import functools

import jax
import jax.numpy as jnp
from jax import lax
from jax.experimental import pallas as pl
from jax.experimental.pallas import tpu as pltpu
from jax.experimental.pallas import tpu_sc as plsc

N_NODES = 10000
D = 128
N_EDGES = 320000

NC = 2
NS = 16
NW = NC * NS
LANES = 16

CHUNK = 80
REAL_W = N_EDGES // NW
NREAL = REAL_W // CHUNK
NCHUNK = 132
NSELF = NCHUNK - NREAL
SELF_W = NSELF * CHUNK
PER_W = NCHUNK * CHUNK
ROWS_PER_TILE = N_NODES // NS
GRP = CHUNK // LANES

_SC_PARAMS = pltpu.CompilerParams(needs_layout_passes=False,
                                  use_tc_tiling_on_sc=False)


def _mesh():
    return plsc.VectorSubcoreMesh(core_axis_name="c", subcore_axis_name="s",
                                  num_cores=NC, num_subcores=NS)


def _wid():
    return lax.axis_index("s") * NC + lax.axis_index("c")


@functools.partial(
    pl.kernel,
    out_type=jax.ShapeDtypeStruct((NW, N_NODES), jnp.float32),
    mesh=_mesh(),
    compiler_params=_SC_PARAMS,
    scratch_types=[
        pltpu.VMEM((REAL_W,), jnp.int32),
        pltpu.VMEM((REAL_W,), jnp.float32),
        pltpu.VMEM((N_NODES,), jnp.float32),
    ],
)
def _deg_kernel(col_hbm, ew_hbm, out_hbm, col_v, ew_v, deg_v):
    w = _wid()
    pltpu.sync_copy(col_hbm.at[w], col_v)
    pltpu.sync_copy(ew_hbm.at[w], ew_v)

    def zbody(i, _):
        deg_v[pl.ds(i * LANES, LANES)] = jnp.zeros((LANES,), jnp.float32)
        return 0

    lax.fori_loop(0, N_NODES // LANES, zbody, 0)

    def ebody(t, _):
        sl = pl.ds(t * LANES, LANES)
        plsc.addupdate_scatter(deg_v, [col_v[sl]], ew_v[sl])
        return 0

    lax.fori_loop(0, REAL_W // LANES, ebody, 0)
    pltpu.sync_copy(deg_v, out_hbm.at[w])


def _tc_pre_body(x_ref, w1_ref, degp_ref, h_ref, dis_ref):
    deg = jnp.sum(degp_ref[...], axis=0, keepdims=True) + 1.0
    dis_ref[...] = lax.rsqrt(deg)
    h_ref[...] = jnp.dot(x_ref[...], w1_ref[...],
                         preferred_element_type=jnp.float32)


_tc_pre = pl.pallas_call(
    _tc_pre_body,
    out_shape=(
        jax.ShapeDtypeStruct((N_NODES, D), jnp.float32),
        jax.ShapeDtypeStruct((1, N_NODES), jnp.float32),
    ),
)


@functools.partial(
    pl.kernel,
    out_type=jax.ShapeDtypeStruct((NW, NCHUNK, 3, CHUNK), jnp.int32),
    mesh=_mesh(),
    compiler_params=_SC_PARAMS,
    scratch_types=[
        pltpu.VMEM((NCHUNK, 3, CHUNK), jnp.int32),
        pltpu.VMEM((N_NODES,), jnp.float32),
    ],
)
def _edata_kernel(row_hbm, col_hbm, ewb_hbm, dis_hbm, out_hbm, obuf, dis_v):
    w = _wid()
    pltpu.sync_copy(row_hbm.at[w], obuf.at[pl.ds(0, NREAL), pl.ds(0, 1)])
    pltpu.sync_copy(col_hbm.at[w], obuf.at[pl.ds(0, NREAL), pl.ds(1, 1)])
    pltpu.sync_copy(ewb_hbm.at[w], obuf.at[pl.ds(0, NREAL), pl.ds(2, 1)])
    pltpu.sync_copy(dis_hbm, dis_v)

    base = w * SELF_W
    one = jnp.full((LANES,), 1.0, jnp.float32)
    zero = jnp.zeros((LANES,), jnp.float32)
    for j in range(NSELF):
        for k in range(GRP):
            ids = (base + j * CHUNK + k * LANES
                   + lax.iota(jnp.int32, LANES))
            valid = ids < N_NODES
            ids_eff = jnp.where(valid, ids, ids - N_NODES)
            sl = pl.ds(k * LANES, LANES)
            obuf[NREAL + j, 0, sl] = ids_eff
            obuf[NREAL + j, 1, sl] = ids_eff
            obuf[NREAL + j, 2, sl] = plsc.bitcast(
                jnp.where(valid, one, zero), jnp.int32)

    def nbody(t, _):
        j = t // GRP
        k = t % GRP
        sl = pl.ds(k * LANES, LANES)
        dr = plsc.load_gather(dis_v, [obuf[j, 0, sl]])
        dc = plsc.load_gather(dis_v, [obuf[j, 1, sl]])
        ew = plsc.bitcast(obuf[j, 2, sl], jnp.float32)
        obuf[j, 2, sl] = plsc.bitcast(dr * ew * dc, jnp.int32)
        return 0

    lax.fori_loop(0, NCHUNK * GRP, nbody, 0)
    pltpu.sync_copy(obuf, out_hbm.at[w])


def _layer_body(h_hbm, ed_hbm, part_hbm,
                eb, gb0, gb1, gb2, acc,
                sh0, sh1, sh2, ss0, ss1, ss2,
                se0, se1, se2, se3, se4, se5):
    core = lax.axis_index("c")
    sub = lax.axis_index("s")
    w = sub * NC + core

    def zb(i, _):
        gb0[i // (D // LANES), pl.ds((i % (D // LANES)) * LANES, LANES)] = (
            jnp.zeros((LANES,), jnp.float32))
        return 0

    lax.fori_loop(0, CHUNK * (D // LANES), zb, 0)
    zbase = sub * ROWS_PER_TILE
    for z in range(ROWS_PER_TILE // CHUNK):
        pltpu.sync_copy(gb0, acc.at[pl.ds(zbase + z * CHUNK, CHUNK)])
    zrem = ROWS_PER_TILE % CHUNK
    if zrem:
        pltpu.sync_copy(
            gb0.at[pl.ds(0, zrem)],
            acc.at[pl.ds(zbase + (ROWS_PER_TILE // CHUNK) * CHUNK, zrem)])
    plsc.subcore_barrier()

    gbs = (gb0, gb1, gb2)
    shs = (sh0, sh1, sh2)
    sss = (ss0, ss1, ss2)
    ses = (se0, se1, se2, se3, se4, se5)

    def scale(gb, m):
        @plsc.parallel_loop(0, CHUNK, 1, unroll=8)
        def _(e):
            nv = plsc.bitcast(
                plsc.load_gather(eb.at[m, 2],
                                 [jnp.full((LANES,), e, jnp.int32)]),
                jnp.float32)
            for b in range(D // LANES):
                sl = pl.ds(b * LANES, LANES)
                gb[e, sl] = gb[e, sl] * nv

    def ecopy(c, m):
        pltpu.async_copy(ed_hbm.at[w, c], eb.at[m], ses[m])

    def ewait(m):
        pltpu.make_async_copy(ed_hbm.at[w, 0], eb.at[m], ses[m]).wait()

    for c in range(5):
        ecopy(c, c)
    ewait(0)
    pltpu.async_copy(h_hbm.at[eb.at[0, 0]], gb0, sh0)
    ewait(1)
    pltpu.async_copy(h_hbm.at[eb.at[1, 0]], gb1, sh1)

    def hexa(i, _):
        for j in range(6):
            c = i * 6 + j
            k = j % 3
            m = j
            gb, sh, ss = gbs[k], shs[k], sss[k]
            kp = (k + 2) % 3
            pltpu.make_async_copy(h_hbm.at[eb.at[m, 0]], gb, sh).wait()
            scale(gb, m)
            pltpu.async_copy(gb, acc.at[eb.at[m, 1]], ss, add=True)

            def drain_prev():
                pltpu.make_async_copy(
                    gbs[kp], acc.at[eb.at[(m + 5) % 6, 1]], sss[kp]).wait()

            if j == 0:
                pl.when(i > 0)(drain_prev)
            else:
                drain_prev()

            @pl.when(c + 5 < NCHUNK)
            def _():
                ecopy(c + 5, (m + 5) % 6)

            @pl.when(c + 2 < NCHUNK)
            def _():
                ewait((m + 2) % 6)
                pltpu.async_copy(h_hbm.at[eb.at[(m + 2) % 6, 0]],
                                 gbs[kp], shs[kp])

        return 0

    lax.fori_loop(0, NCHUNK // 6, hexa, 0)
    pltpu.make_async_copy(gbs[(NCHUNK - 1) % 3],
                          acc.at[eb.at[(NCHUNK - 1) % 6, 1]],
                          sss[(NCHUNK - 1) % 3]).wait()
    plsc.subcore_barrier()

    rbase = sub * ROWS_PER_TILE
    pltpu.sync_copy(acc.at[pl.ds(rbase, ROWS_PER_TILE)],
                    part_hbm.at[core, pl.ds(rbase, ROWS_PER_TILE)])


_layer = pl.kernel(
    _layer_body,
    out_type=jax.ShapeDtypeStruct((NC, N_NODES, D), jnp.float32),
    mesh=_mesh(),
    compiler_params=_SC_PARAMS,
    scratch_types=[
        pltpu.VMEM((6, 3, CHUNK), jnp.int32),
        pltpu.VMEM((CHUNK, D), jnp.float32),
        pltpu.VMEM((CHUNK, D), jnp.float32),
        pltpu.VMEM((CHUNK, D), jnp.float32),
        pltpu.VMEM_SHARED((N_NODES, D), jnp.float32),
        pltpu.SemaphoreType.DMA,
        pltpu.SemaphoreType.DMA,
        pltpu.SemaphoreType.DMA,
        pltpu.SemaphoreType.DMA,
        pltpu.SemaphoreType.DMA,
        pltpu.SemaphoreType.DMA,
        pltpu.SemaphoreType.DMA,
        pltpu.SemaphoreType.DMA,
        pltpu.SemaphoreType.DMA,
        pltpu.SemaphoreType.DMA,
        pltpu.SemaphoreType.DMA,
        pltpu.SemaphoreType.DMA,
    ],
)


def _tc_mid_body(p_ref, b_ref, w2_ref, out_ref):
    s = p_ref[0] + p_ref[1] + b_ref[...]
    out_ref[...] = jnp.dot(jnp.maximum(s, 0.0), w2_ref[...],
                           preferred_element_type=jnp.float32)


_tc_mid = pl.pallas_call(
    _tc_mid_body,
    out_shape=jax.ShapeDtypeStruct((N_NODES, D), jnp.float32),
)


def _tc_post_body(p_ref, b_ref, out_ref):
    out_ref[...] = p_ref[0] + p_ref[1] + b_ref[...]


_tc_post = pl.pallas_call(
    _tc_post_body,
    out_shape=jax.ShapeDtypeStruct((N_NODES, D), jnp.float32),
)


def kernel(x, edge_index, edge_weight, W1, b1, W2, b2):
    ei = edge_index.astype(jnp.int32)
    row2 = ei[0].reshape(NW, NREAL, 1, CHUNK)
    col2 = ei[1].reshape(NW, NREAL, 1, CHUNK)
    ewb2 = lax.bitcast_convert_type(
        edge_weight.astype(jnp.float32), jnp.int32
    ).reshape(NW, NREAL, 1, CHUNK)
    colf = ei[1].reshape(NW, REAL_W)
    ewf = edge_weight.astype(jnp.float32).reshape(NW, REAL_W)

    degp = _deg_kernel(colf, ewf)
    h1, dis2d = _tc_pre(x, W1, degp)
    dis = dis2d.reshape(N_NODES)
    edata = _edata_kernel(row2, col2, ewb2, dis)

    p1 = _layer(h1, edata)
    h2 = _tc_mid(p1, b1.reshape(1, D), W2)

    p2 = _layer(h2, edata)
    out = _tc_post(p2, b2.reshape(1, D))
    return out


# speedup vs baseline: 1.1574x; 1.0116x over previous
import functools

import jax
import jax.numpy as jnp
from jax import lax
from jax.experimental import pallas as pl
from jax.experimental.pallas import tpu as pltpu
from jax.experimental.pallas import tpu_sc as plsc

N_NODES = 10000
D = 128
N_EDGES = 320000

NC = 2
NS = 16
NW = NC * NS
LANES = 16

CHUNK = 80
REAL_W = N_EDGES // NW
NREAL = REAL_W // CHUNK
NCHUNK = 132
NSELF = NCHUNK - NREAL
SELF_W = NSELF * CHUNK
PER_W = NCHUNK * CHUNK
ROWS_PER_TILE = N_NODES // NS
GRP = CHUNK // LANES

_SC_PARAMS = pltpu.CompilerParams(needs_layout_passes=False,
                                  use_tc_tiling_on_sc=False)


def _mesh():
    return plsc.VectorSubcoreMesh(core_axis_name="c", subcore_axis_name="s",
                                  num_cores=NC, num_subcores=NS)


def _wid():
    return lax.axis_index("s") * NC + lax.axis_index("c")


@functools.partial(
    pl.kernel,
    out_type=jax.ShapeDtypeStruct((NW, N_NODES), jnp.float32),
    mesh=_mesh(),
    compiler_params=_SC_PARAMS,
    scratch_types=[
        pltpu.VMEM((REAL_W,), jnp.int32),
        pltpu.VMEM((REAL_W,), jnp.float32),
        pltpu.VMEM((N_NODES,), jnp.float32),
    ],
)
def _deg_kernel(col_hbm, ew_hbm, out_hbm, col_v, ew_v, deg_v):
    w = _wid()
    pltpu.sync_copy(col_hbm.at[w], col_v)
    pltpu.sync_copy(ew_hbm.at[w], ew_v)

    def zbody(i, _):
        deg_v[pl.ds(i * LANES, LANES)] = jnp.zeros((LANES,), jnp.float32)
        return 0

    lax.fori_loop(0, N_NODES // LANES, zbody, 0)

    def ebody(t, _):
        sl = pl.ds(t * LANES, LANES)
        plsc.addupdate_scatter(deg_v, [col_v[sl]], ew_v[sl])
        return 0

    lax.fori_loop(0, REAL_W // LANES, ebody, 0)
    pltpu.sync_copy(deg_v, out_hbm.at[w])


def _tc_h_body(x_ref, w1_ref, h_ref):
    h_ref[...] = jnp.dot(x_ref[...], w1_ref[...],
                         preferred_element_type=jnp.float32)


_tc_h = pl.pallas_call(
    _tc_h_body,
    out_shape=jax.ShapeDtypeStruct((N_NODES, D), jnp.float32),
)


def _tc_dis_body(degp_ref, dis_ref):
    deg = jnp.sum(degp_ref[...], axis=0, keepdims=True) + 1.0
    dis_ref[...] = lax.rsqrt(deg)


_tc_dis = pl.pallas_call(
    _tc_dis_body,
    out_shape=jax.ShapeDtypeStruct((1, N_NODES), jnp.float32),
)


@functools.partial(
    pl.kernel,
    out_type=jax.ShapeDtypeStruct((NW, NCHUNK, 3, CHUNK), jnp.int32),
    mesh=_mesh(),
    compiler_params=_SC_PARAMS,
    scratch_types=[
        pltpu.VMEM((NCHUNK, 3, CHUNK), jnp.int32),
        pltpu.VMEM((N_NODES,), jnp.float32),
    ],
)
def _edata_kernel(row_hbm, col_hbm, ewb_hbm, dis_hbm, out_hbm, obuf, dis_v):
    w = _wid()
    pltpu.sync_copy(row_hbm.at[w], obuf.at[pl.ds(0, NREAL), pl.ds(0, 1)])
    pltpu.sync_copy(col_hbm.at[w], obuf.at[pl.ds(0, NREAL), pl.ds(1, 1)])
    pltpu.sync_copy(ewb_hbm.at[w], obuf.at[pl.ds(0, NREAL), pl.ds(2, 1)])
    pltpu.sync_copy(dis_hbm, dis_v)

    base = w * SELF_W
    one = jnp.full((LANES,), 1.0, jnp.float32)
    zero = jnp.zeros((LANES,), jnp.float32)
    for j in range(NSELF):
        for k in range(GRP):
            ids = (base + j * CHUNK + k * LANES
                   + lax.iota(jnp.int32, LANES))
            valid = ids < N_NODES
            ids_eff = jnp.where(valid, ids, ids - N_NODES)
            sl = pl.ds(k * LANES, LANES)
            obuf[NREAL + j, 0, sl] = ids_eff
            obuf[NREAL + j, 1, sl] = ids_eff
            obuf[NREAL + j, 2, sl] = plsc.bitcast(
                jnp.where(valid, one, zero), jnp.int32)

    def nbody(t, _):
        j = t // GRP
        k = t % GRP
        sl = pl.ds(k * LANES, LANES)
        dr = plsc.load_gather(dis_v, [obuf[j, 0, sl]])
        dc = plsc.load_gather(dis_v, [obuf[j, 1, sl]])
        ew = plsc.bitcast(obuf[j, 2, sl], jnp.float32)
        obuf[j, 2, sl] = plsc.bitcast(dr * ew * dc, jnp.int32)
        return 0

    lax.fori_loop(0, NCHUNK * GRP, nbody, 0)
    pltpu.sync_copy(obuf, out_hbm.at[w])


def _layer_body(h_hbm, ed_hbm, part_hbm,
                eb, gb0, gb1, gb2, acc,
                sh0, sh1, sh2, ss0, ss1, ss2,
                se0, se1, se2, se3, se4, se5):
    core = lax.axis_index("c")
    sub = lax.axis_index("s")
    w = sub * NC + core

    def zb(i, _):
        gb0[i // (D // LANES), pl.ds((i % (D // LANES)) * LANES, LANES)] = (
            jnp.zeros((LANES,), jnp.float32))
        return 0

    lax.fori_loop(0, CHUNK * (D // LANES), zb, 0)
    zbase = sub * ROWS_PER_TILE
    for z in range(ROWS_PER_TILE // CHUNK):
        pltpu.sync_copy(gb0, acc.at[pl.ds(zbase + z * CHUNK, CHUNK)])
    zrem = ROWS_PER_TILE % CHUNK
    if zrem:
        pltpu.sync_copy(
            gb0.at[pl.ds(0, zrem)],
            acc.at[pl.ds(zbase + (ROWS_PER_TILE // CHUNK) * CHUNK, zrem)])
    plsc.subcore_barrier()

    gbs = (gb0, gb1, gb2)
    shs = (sh0, sh1, sh2)
    sss = (ss0, ss1, ss2)
    ses = (se0, se1, se2, se3, se4, se5)

    def scale(gb, m):
        @plsc.parallel_loop(0, CHUNK, 1, unroll=8)
        def _(e):
            nv = plsc.bitcast(
                plsc.load_gather(eb.at[m, 2],
                                 [jnp.full((LANES,), e, jnp.int32)]),
                jnp.float32)
            for b in range(D // LANES):
                sl = pl.ds(b * LANES, LANES)
                gb[e, sl] = gb[e, sl] * nv

    def ecopy(c, m):
        pltpu.async_copy(ed_hbm.at[w, c], eb.at[m], ses[m])

    def ewait(m):
        pltpu.make_async_copy(ed_hbm.at[w, 0], eb.at[m], ses[m]).wait()

    for c in range(5):
        ecopy(c, c)
    ewait(0)
    pltpu.async_copy(h_hbm.at[eb.at[0, 0]], gb0, sh0)
    ewait(1)
    pltpu.async_copy(h_hbm.at[eb.at[1, 0]], gb1, sh1)

    def hexa(i, _):
        for j in range(6):
            c = i * 6 + j
            k = j % 3
            m = j
            gb, sh, ss = gbs[k], shs[k], sss[k]
            kp = (k + 2) % 3
            pltpu.make_async_copy(h_hbm.at[eb.at[m, 0]], gb, sh).wait()
            scale(gb, m)
            pltpu.async_copy(gb, acc.at[eb.at[m, 1]], ss, add=True)

            def drain_prev():
                pltpu.make_async_copy(
                    gbs[kp], acc.at[eb.at[(m + 5) % 6, 1]], sss[kp]).wait()

            if j == 0:
                pl.when(i > 0)(drain_prev)
            else:
                drain_prev()

            @pl.when(c + 5 < NCHUNK)
            def _():
                ecopy(c + 5, (m + 5) % 6)

            @pl.when(c + 2 < NCHUNK)
            def _():
                ewait((m + 2) % 6)
                pltpu.async_copy(h_hbm.at[eb.at[(m + 2) % 6, 0]],
                                 gbs[kp], shs[kp])

        return 0

    lax.fori_loop(0, NCHUNK // 6, hexa, 0)
    pltpu.make_async_copy(gbs[(NCHUNK - 1) % 3],
                          acc.at[eb.at[(NCHUNK - 1) % 6, 1]],
                          sss[(NCHUNK - 1) % 3]).wait()
    plsc.subcore_barrier()

    rbase = sub * ROWS_PER_TILE
    pltpu.sync_copy(acc.at[pl.ds(rbase, ROWS_PER_TILE)],
                    part_hbm.at[core, pl.ds(rbase, ROWS_PER_TILE)])


_layer = pl.kernel(
    _layer_body,
    out_type=jax.ShapeDtypeStruct((NC, N_NODES, D), jnp.float32),
    mesh=_mesh(),
    compiler_params=_SC_PARAMS,
    scratch_types=[
        pltpu.VMEM((6, 3, CHUNK), jnp.int32),
        pltpu.VMEM((CHUNK, D), jnp.float32),
        pltpu.VMEM((CHUNK, D), jnp.float32),
        pltpu.VMEM((CHUNK, D), jnp.float32),
        pltpu.VMEM_SHARED((N_NODES, D), jnp.float32),
        pltpu.SemaphoreType.DMA,
        pltpu.SemaphoreType.DMA,
        pltpu.SemaphoreType.DMA,
        pltpu.SemaphoreType.DMA,
        pltpu.SemaphoreType.DMA,
        pltpu.SemaphoreType.DMA,
        pltpu.SemaphoreType.DMA,
        pltpu.SemaphoreType.DMA,
        pltpu.SemaphoreType.DMA,
        pltpu.SemaphoreType.DMA,
        pltpu.SemaphoreType.DMA,
        pltpu.SemaphoreType.DMA,
    ],
)


def _tc_mid_body(p_ref, b_ref, w2_ref, out_ref):
    s = p_ref[0] + p_ref[1] + b_ref[...]
    out_ref[...] = jnp.dot(jnp.maximum(s, 0.0), w2_ref[...],
                           preferred_element_type=jnp.float32)


_tc_mid = pl.pallas_call(
    _tc_mid_body,
    out_shape=jax.ShapeDtypeStruct((N_NODES, D), jnp.float32),
)


def _tc_post_body(p_ref, b_ref, out_ref):
    out_ref[...] = p_ref[0] + p_ref[1] + b_ref[...]


_tc_post = pl.pallas_call(
    _tc_post_body,
    out_shape=jax.ShapeDtypeStruct((N_NODES, D), jnp.float32),
)


def kernel(x, edge_index, edge_weight, W1, b1, W2, b2):
    ei = edge_index.astype(jnp.int32)
    row2 = ei[0].reshape(NW, NREAL, 1, CHUNK)
    col2 = ei[1].reshape(NW, NREAL, 1, CHUNK)
    ewb2 = lax.bitcast_convert_type(
        edge_weight.astype(jnp.float32), jnp.int32
    ).reshape(NW, NREAL, 1, CHUNK)
    colf = ei[1].reshape(NW, REAL_W)
    ewf = edge_weight.astype(jnp.float32).reshape(NW, REAL_W)

    h1 = _tc_h(x, W1)
    degp = _deg_kernel(colf, ewf)
    dis = _tc_dis(degp).reshape(N_NODES)
    edata = _edata_kernel(row2, col2, ewb2, dis)

    p1 = _layer(h1, edata)
    h2 = _tc_mid(p1, b1.reshape(1, D), W2)

    p2 = _layer(h2, edata)
    out = _tc_post(p2, b2.reshape(1, D))
    return out
